# baseline (speedup 1.0000x reference)
import functools

import jax
import jax.numpy as jnp
from jax.experimental import pallas as pl
from jax.experimental.pallas import tpu as pltpu

_M = 6
_A = 4
_QH = 2
_HD = 128
_NEG = -1e9


def _route_body(x_ref, wr_ref, kvi_ref, kvw_ref, mem_ref, aux_ref,
                p_acc, f_acc, e_acc, *, bs, s_total):
    i = pl.program_id(0)
    nsteps = pl.num_programs(0)

    @pl.when(i == 0)
    def _init():
        p_acc[...] = jnp.zeros_like(p_acc)
        f_acc[...] = jnp.zeros_like(f_acc)
        e_acc[...] = jnp.zeros_like(e_acc)

    x = x_ref[...]
    logits = jax.lax.dot_general(
        x, wr_ref[...], (((1,), (0,)), ((), ())),
        preferred_element_type=jnp.float32)
    lane = jax.lax.broadcasted_iota(jnp.int32, (bs, 128), 1)
    valid = lane < _M
    lm = jnp.where(valid, logits, -1e30)

    mx = jnp.max(lm, axis=1, keepdims=True)
    ex = jnp.where(valid, jnp.exp(lm - mx), 0.0)
    den = jnp.sum(ex, axis=1, keepdims=True)
    soft = ex / den
    p_acc[...] += jnp.sum(soft, axis=0, keepdims=True)
    ent_rows = -jnp.sum(
        jnp.where(valid, soft * jnp.log(soft + 1e-8), 0.0), axis=1,
        keepdims=True)
    e_acc[...] += jnp.where(
        jax.lax.broadcasted_iota(jnp.int32, (1, 128), 1) == 0,
        jnp.sum(ent_rows), 0.0)

    work = lm
    idxs = []
    vals = []
    for _ in range(_A):
        cur = jnp.max(work, axis=1, keepdims=True)
        hit = work == cur
        idx = jnp.min(jnp.where(hit, lane, 128), axis=1, keepdims=True)
        idxs.append(idx)
        vals.append(cur)
        work = jnp.where(lane == idx, -1e30, work)

    f_acc[...] += jnp.sum((lane == idxs[0]).astype(jnp.float32), axis=0,
                          keepdims=True)

    es = [jnp.exp(v - vals[0]) for v in vals]
    wden = es[0] + es[1] + es[2] + es[3]

    kvi = jnp.zeros((bs, 128), jnp.int32)
    kvw = jnp.zeros((bs, 128), jnp.float32)
    for a in range(_A):
        kvi = jnp.where(lane == a, idxs[a], kvi)
        kvw = jnp.where(lane == a, es[a] / wden, kvw)
    kvi_ref[...] = kvi
    kvw_ref[...] = kvw

    lane8 = jax.lax.broadcasted_iota(jnp.int32, (bs, 8), 1)
    sel = jnp.zeros((bs, 8), jnp.bool_)
    for a in range(_A):
        sel = sel | (lane8 == idxs[a])
    mem_ref[...] = jnp.where(sel, 0.0, _NEG)

    @pl.when(i == nsteps - 1)
    def _fin():
        s_f = jnp.float32(s_total)
        balance = _M * jnp.sum(p_acc[...] * f_acc[...]) / (s_f * s_f)
        ent_mean = jnp.sum(e_acc[...]) / s_f
        aux_ref[...] = jnp.broadcast_to(0.01 * balance - 0.01 * ent_mean,
                                        (1, 1))


def _qkv_body(x_ref, wq_ref, wk_ref, wv_ref, cos_ref, sin_ref,
              q_ref, k_ref, v_ref):
    x = x_ref[...]
    cos = cos_ref[...]
    sin = sin_ref[...]

    def rope(h):
        rot = jnp.concatenate([-h[:, _HD // 2:], h[:, :_HD // 2]], axis=1)
        return h * cos + rot * sin

    xq = jax.lax.dot_general(x, wq_ref[...], (((1,), (0,)), ((), ())),
                             preferred_element_type=jnp.float32)
    for h in range(_M * _QH):
        q_ref[h] = rope(xq[:, h * _HD:(h + 1) * _HD])
    xk = jax.lax.dot_general(x, wk_ref[...], (((1,), (0,)), ((), ())),
                             preferred_element_type=jnp.float32)
    for m in range(_M):
        k_ref[m] = rope(xk[:, m * _HD:(m + 1) * _HD])
    xv = jax.lax.dot_general(x, wv_ref[...], (((1,), (0,)), ((), ())),
                             preferred_element_type=jnp.float32)
    for m in range(_M):
        v_ref[m] = xv[:, m * _HD:(m + 1) * _HD]


def _attn_body(q_ref, k_ref, v_ref, mem_ref, o_ref, *, bq, bk, scale):
    qb = pl.program_id(1)
    row0 = jax.lax.broadcasted_iota(jnp.int32, (bq, bk), 0) + qb * bq
    col0 = jax.lax.broadcasted_iota(jnp.int32, (bq, bk), 1)

    for h in range(_QH):
        q = q_ref[h]

        def body(kt, carry):
            acc, mx, l = carry
            kk = k_ref[0, pl.ds(kt * bk, bk), :]
            vv = v_ref[0, pl.ds(kt * bk, bk), :]
            bias = mem_ref[0, :, pl.ds(kt * bk, bk)]
            s = jax.lax.dot_general(
                q, kk, (((1,), (1,)), ((), ())),
                preferred_element_type=jnp.float32) * scale
            s = jnp.where(col0 + kt * bk <= row0, s + bias, _NEG)
            new_mx = jnp.maximum(mx, jnp.max(s, axis=1, keepdims=True))
            p = jnp.exp(s - new_mx)
            corr = jnp.exp(mx - new_mx)
            l = l * corr + jnp.sum(p, axis=1, keepdims=True)
            acc = acc * corr + jax.lax.dot_general(
                p, vv, (((1,), (0,)), ((), ())),
                preferred_element_type=jnp.float32)
            return acc, new_mx, l

        acc, _, l = jax.lax.fori_loop(
            0, qb * bq // bk + 1, body,
            (jnp.zeros((bq, _HD), jnp.float32),
             jnp.full((bq, 1), -1e30, jnp.float32),
             jnp.zeros((bq, 1), jnp.float32)))
        o_ref[0, :, h * _HD:(h + 1) * _HD] = acc / l


def _out_body(o_ref, kvi_ref, kvw_ref, wo_ref, out_ref, *, bs):
    kvi = kvi_ref[...]
    kvw = kvw_ref[...]
    acc = jnp.zeros((bs, 1024), jnp.float32)
    for a in range(_A):
        ia = kvi[:, a:a + 1]
        wa = kvw[:, a:a + 1]
        g = jnp.zeros((bs, _QH * _HD), jnp.float32)
        for m in range(_M):
            g = g + jnp.where(ia == m, 1.0, 0.0) * o_ref[m]
        g = g * wa
        acc = acc + jax.lax.dot_general(
            g, wo_ref[a * _QH * _HD:(a + 1) * _QH * _HD, :],
            (((1,), (0,)), ((), ())), preferred_element_type=jnp.float32)
    out_ref[...] = acc


def kernel(x, Wq, Wk, Wv, Wr, Wo):
    b, s, e = x.shape
    xs = x.reshape(s, e)
    bs = 256
    bq = 256
    scale = _HD ** -0.5

    wr_pad = jnp.zeros((e, 128), jnp.float32).at[:, :_M].set(Wr)

    kvi, kvw, member, aux = pl.pallas_call(
        functools.partial(_route_body, bs=bs, s_total=s),
        grid=(s // bs,),
        in_specs=[
            pl.BlockSpec((bs, e), lambda i: (i, 0)),
            pl.BlockSpec((e, 128), lambda i: (0, 0)),
        ],
        out_specs=[
            pl.BlockSpec((bs, 128), lambda i: (i, 0)),
            pl.BlockSpec((bs, 128), lambda i: (i, 0)),
            pl.BlockSpec((bs, 8), lambda i: (i, 0)),
            pl.BlockSpec((1, 1), lambda i: (0, 0)),
        ],
        out_shape=[
            jax.ShapeDtypeStruct((s, 128), jnp.int32),
            jax.ShapeDtypeStruct((s, 128), jnp.float32),
            jax.ShapeDtypeStruct((s, 8), jnp.float32),
            jax.ShapeDtypeStruct((1, 1), jnp.float32),
        ],
        scratch_shapes=[
            pltpu.VMEM((1, 128), jnp.float32),
            pltpu.VMEM((1, 128), jnp.float32),
            pltpu.VMEM((1, 128), jnp.float32),
        ],
    )(xs, wr_pad)

    inv = 1.0 / (10000.0 ** (jnp.arange(0, _HD, 2, dtype=jnp.float32) / _HD))
    t = jnp.arange(s, dtype=jnp.float32)
    fr = jnp.outer(t, inv)
    emb = jnp.concatenate([fr, fr], axis=-1)
    cos_t, sin_t = jnp.cos(emb), jnp.sin(emb)

    q, k, v = pl.pallas_call(
        _qkv_body,
        grid=(s // bs,),
        in_specs=[
            pl.BlockSpec((bs, e), lambda i: (i, 0)),
            pl.BlockSpec((e, _M * _QH * _HD), lambda i: (0, 0)),
            pl.BlockSpec((e, _M * _HD), lambda i: (0, 0)),
            pl.BlockSpec((e, _M * _HD), lambda i: (0, 0)),
            pl.BlockSpec((bs, _HD), lambda i: (i, 0)),
            pl.BlockSpec((bs, _HD), lambda i: (i, 0)),
        ],
        out_specs=[
            pl.BlockSpec((_M * _QH, bs, _HD), lambda i: (0, i, 0)),
            pl.BlockSpec((_M, bs, _HD), lambda i: (0, i, 0)),
            pl.BlockSpec((_M, bs, _HD), lambda i: (0, i, 0)),
        ],
        out_shape=[
            jax.ShapeDtypeStruct((_M * _QH, s, _HD), jnp.float32),
            jax.ShapeDtypeStruct((_M, s, _HD), jnp.float32),
            jax.ShapeDtypeStruct((_M, s, _HD), jnp.float32),
        ],
    )(xs, Wq, Wk, Wv, cos_t, sin_t)

    member_t = member.T.reshape(8, 1, s)

    o = pl.pallas_call(
        functools.partial(_attn_body, bq=bq, bk=bq, scale=scale),
        grid=(_M, s // bq),
        in_specs=[
            pl.BlockSpec((_QH, bq, _HD), lambda m, i: (m, i, 0)),
            pl.BlockSpec((1, s, _HD), lambda m, i: (m, 0, 0)),
            pl.BlockSpec((1, s, _HD), lambda m, i: (m, 0, 0)),
            pl.BlockSpec((1, 1, s), lambda m, i: (m, 0, 0)),
        ],
        out_specs=pl.BlockSpec((1, bq, _QH * _HD), lambda m, i: (m, i, 0)),
        out_shape=jax.ShapeDtypeStruct((_M, s, _QH * _HD), jnp.float32),
    )(q, k, v, member_t)

    out = pl.pallas_call(
        functools.partial(_out_body, bs=bs),
        grid=(s // bs,),
        in_specs=[
            pl.BlockSpec((_M, bs, _QH * _HD), lambda i: (0, i, 0)),
            pl.BlockSpec((bs, 128), lambda i: (i, 0)),
            pl.BlockSpec((bs, 128), lambda i: (i, 0)),
            pl.BlockSpec((_A * _QH * _HD, e), lambda i: (0, 0)),
        ],
        out_specs=pl.BlockSpec((bs, e), lambda i: (i, 0)),
        out_shape=jax.ShapeDtypeStruct((s, e), jnp.float32),
    )(o, kvi, kvw, Wo)

    return out.reshape(b, s, e), aux.reshape(())


# speedup vs baseline: 1.6881x; 1.6881x over previous
import functools

import jax
import jax.numpy as jnp
from jax.experimental import pallas as pl
from jax.experimental.pallas import tpu as pltpu

_M = 6
_A = 4
_QH = 2
_HD = 128
_NEG = -1e9


def _route_body(x_ref, wr_ref, kvi_ref, kvw_ref, mem_ref, aux_ref,
                p_acc, f_acc, e_acc, *, bs, s_total):
    i = pl.program_id(0)
    nsteps = pl.num_programs(0)

    @pl.when(i == 0)
    def _init():
        p_acc[...] = jnp.zeros_like(p_acc)
        f_acc[...] = jnp.zeros_like(f_acc)
        e_acc[...] = jnp.zeros_like(e_acc)

    x = x_ref[...]
    logits = jax.lax.dot_general(
        x, wr_ref[...], (((1,), (0,)), ((), ())),
        preferred_element_type=jnp.float32)
    lane = jax.lax.broadcasted_iota(jnp.int32, (bs, 128), 1)
    valid = lane < _M
    lm = jnp.where(valid, logits, -1e30)

    mx = jnp.max(lm, axis=1, keepdims=True)
    ex = jnp.where(valid, jnp.exp(lm - mx), 0.0)
    den = jnp.sum(ex, axis=1, keepdims=True)
    soft = ex / den
    p_acc[...] += jnp.sum(soft, axis=0, keepdims=True)
    ent_rows = -jnp.sum(
        jnp.where(valid, soft * jnp.log(soft + 1e-8), 0.0), axis=1,
        keepdims=True)
    e_acc[...] += jnp.where(
        jax.lax.broadcasted_iota(jnp.int32, (1, 128), 1) == 0,
        jnp.sum(ent_rows), 0.0)

    work = lm
    idxs = []
    vals = []
    for _ in range(_A):
        cur = jnp.max(work, axis=1, keepdims=True)
        hit = work == cur
        idx = jnp.min(jnp.where(hit, lane, 128), axis=1, keepdims=True)
        idxs.append(idx)
        vals.append(cur)
        work = jnp.where(lane == idx, -1e30, work)

    f_acc[...] += jnp.sum((lane == idxs[0]).astype(jnp.float32), axis=0,
                          keepdims=True)

    es = [jnp.exp(v - vals[0]) for v in vals]
    wden = es[0] + es[1] + es[2] + es[3]

    kvi = jnp.zeros((bs, 128), jnp.int32)
    kvw = jnp.zeros((bs, 128), jnp.float32)
    for a in range(_A):
        kvi = jnp.where(lane == a, idxs[a], kvi)
        kvw = jnp.where(lane == a, es[a] / wden, kvw)
    kvi_ref[...] = kvi
    kvw_ref[...] = kvw

    lane8 = jax.lax.broadcasted_iota(jnp.int32, (bs, 8), 1)
    sel = jnp.zeros((bs, 8), jnp.bool_)
    for a in range(_A):
        sel = sel | (lane8 == idxs[a])
    mem_ref[...] = jnp.where(sel, 0.0, _NEG)

    @pl.when(i == nsteps - 1)
    def _fin():
        s_f = jnp.float32(s_total)
        balance = _M * jnp.sum(p_acc[...] * f_acc[...]) / (s_f * s_f)
        ent_mean = jnp.sum(e_acc[...]) / s_f
        aux_ref[...] = jnp.broadcast_to(0.01 * balance - 0.01 * ent_mean,
                                        (1, 1))


def _qkv_body(x_ref, wq_ref, wk_ref, wv_ref, cos_ref, sin_ref,
              q_ref, k_ref, v_ref):
    x = x_ref[...]
    cos = cos_ref[...]
    sin = sin_ref[...]

    def rope(h):
        rot = jnp.concatenate([-h[:, _HD // 2:], h[:, :_HD // 2]], axis=1)
        return h * cos + rot * sin

    xq = jax.lax.dot_general(x, wq_ref[...], (((1,), (0,)), ((), ())),
                             preferred_element_type=jnp.float32)
    for h in range(_M * _QH):
        q_ref[h] = rope(xq[:, h * _HD:(h + 1) * _HD])
    xk = jax.lax.dot_general(x, wk_ref[...], (((1,), (0,)), ((), ())),
                             preferred_element_type=jnp.float32)
    for m in range(_M):
        k_ref[m] = rope(xk[:, m * _HD:(m + 1) * _HD])
    xv = jax.lax.dot_general(x, wv_ref[...], (((1,), (0,)), ((), ())),
                             preferred_element_type=jnp.float32)
    for m in range(_M):
        v_ref[m] = xv[:, m * _HD:(m + 1) * _HD]


def _attn_body(q_ref, k_ref, v_ref, mem_ref, o_ref, *, bq, sk, q0, scale):
    qb = pl.program_id(1)
    kk = k_ref[0]
    vv = v_ref[0]
    bias = mem_ref[0]
    row = (jax.lax.broadcasted_iota(jnp.int32, (bq, sk), 0)
           + (q0 + qb) * bq)
    col = jax.lax.broadcasted_iota(jnp.int32, (bq, sk), 1)
    causal = col <= row

    for h in range(_QH):
        q = q_ref[h]
        s = jax.lax.dot_general(q, kk, (((1,), (1,)), ((), ())),
                                preferred_element_type=jnp.float32) * scale
        s = jnp.where(causal, s + bias, _NEG)
        mx = jnp.max(s, axis=1, keepdims=True)
        p = jnp.exp(s - mx)
        den = jnp.sum(p, axis=1, keepdims=True)
        o = jax.lax.dot_general(p, vv, (((1,), (0,)), ((), ())),
                                preferred_element_type=jnp.float32) / den
        o_ref[0, :, h * _HD:(h + 1) * _HD] = o


def _out_body(o_ref, kvi_ref, kvw_ref, wo_ref, out_ref, *, bs):
    kvi = kvi_ref[...]
    kvw = kvw_ref[...]
    acc = jnp.zeros((bs, 1024), jnp.float32)
    for a in range(_A):
        ia = kvi[:, a:a + 1]
        wa = kvw[:, a:a + 1]
        g = jnp.zeros((bs, _QH * _HD), jnp.float32)
        for m in range(_M):
            g = g + jnp.where(ia == m, 1.0, 0.0) * o_ref[m]
        g = g * wa
        acc = acc + jax.lax.dot_general(
            g, wo_ref[a * _QH * _HD:(a + 1) * _QH * _HD, :],
            (((1,), (0,)), ((), ())), preferred_element_type=jnp.float32)
    out_ref[...] = acc


def kernel(x, Wq, Wk, Wv, Wr, Wo):
    b, s, e = x.shape
    xs = x.reshape(s, e)
    bs = 256
    bq = 256
    scale = _HD ** -0.5

    wr_pad = jnp.zeros((e, 128), jnp.float32).at[:, :_M].set(Wr)

    kvi, kvw, member, aux = pl.pallas_call(
        functools.partial(_route_body, bs=bs, s_total=s),
        grid=(s // bs,),
        in_specs=[
            pl.BlockSpec((bs, e), lambda i: (i, 0)),
            pl.BlockSpec((e, 128), lambda i: (0, 0)),
        ],
        out_specs=[
            pl.BlockSpec((bs, 128), lambda i: (i, 0)),
            pl.BlockSpec((bs, 128), lambda i: (i, 0)),
            pl.BlockSpec((bs, 8), lambda i: (i, 0)),
            pl.BlockSpec((1, 1), lambda i: (0, 0)),
        ],
        out_shape=[
            jax.ShapeDtypeStruct((s, 128), jnp.int32),
            jax.ShapeDtypeStruct((s, 128), jnp.float32),
            jax.ShapeDtypeStruct((s, 8), jnp.float32),
            jax.ShapeDtypeStruct((1, 1), jnp.float32),
        ],
        scratch_shapes=[
            pltpu.VMEM((1, 128), jnp.float32),
            pltpu.VMEM((1, 128), jnp.float32),
            pltpu.VMEM((1, 128), jnp.float32),
        ],
    )(xs, wr_pad)

    inv = 1.0 / (10000.0 ** (jnp.arange(0, _HD, 2, dtype=jnp.float32) / _HD))
    t = jnp.arange(s, dtype=jnp.float32)
    fr = jnp.outer(t, inv)
    emb = jnp.concatenate([fr, fr], axis=-1)
    cos_t, sin_t = jnp.cos(emb), jnp.sin(emb)

    q, k, v = pl.pallas_call(
        _qkv_body,
        grid=(s // bs,),
        in_specs=[
            pl.BlockSpec((bs, e), lambda i: (i, 0)),
            pl.BlockSpec((e, _M * _QH * _HD), lambda i: (0, 0)),
            pl.BlockSpec((e, _M * _HD), lambda i: (0, 0)),
            pl.BlockSpec((e, _M * _HD), lambda i: (0, 0)),
            pl.BlockSpec((bs, _HD), lambda i: (i, 0)),
            pl.BlockSpec((bs, _HD), lambda i: (i, 0)),
        ],
        out_specs=[
            pl.BlockSpec((_M * _QH, bs, _HD), lambda i: (0, i, 0)),
            pl.BlockSpec((_M, bs, _HD), lambda i: (0, i, 0)),
            pl.BlockSpec((_M, bs, _HD), lambda i: (0, i, 0)),
        ],
        out_shape=[
            jax.ShapeDtypeStruct((_M * _QH, s, _HD), jnp.float32),
            jax.ShapeDtypeStruct((_M, s, _HD), jnp.float32),
            jax.ShapeDtypeStruct((_M, s, _HD), jnp.float32),
        ],
    )(xs, Wq, Wk, Wv, cos_t, sin_t)

    member_t = member.T.reshape(8, 1, s)

    nqb = s // bq
    seg_blocks = 2
    o_parts = []
    for j0 in range(0, nqb, seg_blocks):
        sk = (j0 + seg_blocks) * bq
        o_parts.append(pl.pallas_call(
            functools.partial(_attn_body, bq=bq, sk=sk, q0=j0, scale=scale),
            grid=(_M, seg_blocks),
            in_specs=[
                pl.BlockSpec((_QH, bq, _HD),
                             lambda m, i, j0=j0: (m, j0 + i, 0)),
                pl.BlockSpec((1, sk, _HD), lambda m, i: (m, 0, 0)),
                pl.BlockSpec((1, sk, _HD), lambda m, i: (m, 0, 0)),
                pl.BlockSpec((1, 1, sk), lambda m, i: (m, 0, 0)),
            ],
            out_specs=pl.BlockSpec((1, bq, _QH * _HD), lambda m, i: (m, i, 0)),
            out_shape=jax.ShapeDtypeStruct(
                (_M, seg_blocks * bq, _QH * _HD), jnp.float32),
        )(q, k, v, member_t))
    o = jnp.concatenate(o_parts, axis=1)

    out = pl.pallas_call(
        functools.partial(_out_body, bs=bs),
        grid=(s // bs,),
        in_specs=[
            pl.BlockSpec((_M, bs, _QH * _HD), lambda i: (0, i, 0)),
            pl.BlockSpec((bs, 128), lambda i: (i, 0)),
            pl.BlockSpec((bs, 128), lambda i: (i, 0)),
            pl.BlockSpec((_A * _QH * _HD, e), lambda i: (0, 0)),
        ],
        out_specs=pl.BlockSpec((bs, e), lambda i: (i, 0)),
        out_shape=jax.ShapeDtypeStruct((s, e), jnp.float32),
    )(o, kvi, kvw, Wo)

    return out.reshape(b, s, e), aux.reshape(())
